# Initial kernel scaffold; baseline (speedup 1.0000x reference)
import jax
import jax.numpy as jnp
from jax.experimental import pallas as pl


def kernel(x, table, lora_A, lora_B):
    raise NotImplementedError("write your pallas kernel here")


# speedup vs baseline: 2.9365x; 2.9365x over previous
import functools

import jax
import jax.numpy as jnp
from jax import lax
from jax.experimental import pallas as pl
from jax.experimental.pallas import tpu as pltpu
from jax.experimental.pallas import tpu_sc as plsc

EMBED_DIM = 64
LORA_R = 16
LORA_SCALING = 2.0
LANES = 16
CHUNK = 128


@functools.lru_cache(maxsize=None)
def _make_kernel(n_tokens: int):
    info = plsc.get_sparse_core_info()
    num_cores, num_subcores = info.num_cores, info.num_subcores
    num_workers = num_cores * num_subcores
    per_worker = n_tokens // num_workers
    assert per_worker * num_workers == n_tokens
    assert per_worker % CHUNK == 0
    n_chunks = per_worker // CHUNK
    d_groups = EMBED_DIM // LANES

    mesh = plsc.VectorSubcoreMesh(core_axis_name="c", subcore_axis_name="s")

    @functools.partial(
        pl.kernel,
        mesh=mesh,
        compiler_params=pltpu.CompilerParams(use_tc_tiling_on_sc=False),
        out_type=jax.ShapeDtypeStruct((n_tokens, EMBED_DIM), jnp.float32),
        scratch_types=[
            pltpu.VMEM((CHUNK,), jnp.int32),
            pltpu.VMEM((CHUNK, EMBED_DIM), jnp.float32),
            pltpu.VMEM((CHUNK, LORA_R), jnp.float32),
            pltpu.VMEM((LORA_R, EMBED_DIM), jnp.float32),
            pltpu.SemaphoreType.DMA,
            pltpu.SemaphoreType.DMA,
        ],
    )
    def sc_kernel(table_hbm, at_hbm, bt_hbm, idx_hbm, out_hbm,
                  idx_v, rows_v, a_v, b_v, sem_rows, sem_a):
        wid = lax.axis_index("s") * num_cores + lax.axis_index("c")
        base0 = wid * per_worker

        pltpu.sync_copy(bt_hbm, b_v)
        b_regs = [[b_v[j, pl.ds(d * LANES, LANES)] for d in range(d_groups)]
                  for j in range(LORA_R)]

        def chunk_body(c, carry):
            base = base0 + c * CHUNK
            pltpu.sync_copy(idx_hbm.at[pl.ds(base, CHUNK)], idx_v)
            cp_rows = pltpu.async_copy(table_hbm.at[idx_v], rows_v, sem_rows)
            cp_a = pltpu.async_copy(at_hbm.at[idx_v], a_v, sem_a)
            cp_rows.wait()
            cp_a.wait()

            def tok_body(t, tc):
                accs = [rows_v[t, pl.ds(d * LANES, LANES)]
                        for d in range(d_groups)]
                a_vec = a_v[t, pl.ds(0, LORA_R)]
                for j in range(LORA_R):
                    a_s = a_vec[j]
                    for d in range(d_groups):
                        accs[d] = accs[d] + a_s * b_regs[j][d]
                for d in range(d_groups):
                    rows_v[t, pl.ds(d * LANES, LANES)] = accs[d]
                return tc

            lax.fori_loop(0, CHUNK, tok_body, 0)
            pltpu.sync_copy(rows_v, out_hbm.at[pl.ds(base, CHUNK)])
            return carry

        lax.fori_loop(0, n_chunks, chunk_body, 0)

    return sc_kernel


def kernel(x, table, lora_A, lora_B):
    batch, hist = x.shape
    n_tokens = batch * hist
    xf = x.reshape(-1).astype(jnp.int32)
    a_t = lora_A.T
    b_t = (lora_B * LORA_SCALING).T.astype(jnp.float32)
    out = _make_kernel(n_tokens)(table, a_t, b_t, xf)
    return out.reshape(batch, hist, EMBED_DIM)
